# baseline (speedup 1.0000x reference)
import jax
import jax.numpy as jnp
from jax.experimental import pallas as pl


_BLK = 512


def _add_kernel(nrep, x_ref, p_ref, o_ref):
    p = p_ref[...]
    o_ref[...] = x_ref[...] + jnp.concatenate([p] * nrep, axis=1)


def kernel(x, pos_embed):
    S, B, D = x.shape
    x2 = x.reshape(S, B * D)
    blk = _BLK if S % _BLK == 0 else S
    out2 = pl.pallas_call(
        lambda xr, pr, orf: _add_kernel(B, xr, pr, orf),
        grid=(S // blk,),
        in_specs=[
            pl.BlockSpec((blk, B * D), lambda i: (i, 0)),
            pl.BlockSpec((blk, D), lambda i: (i, 0)),
        ],
        out_specs=pl.BlockSpec((blk, B * D), lambda i: (i, 0)),
        out_shape=jax.ShapeDtypeStruct((S, B * D), x.dtype),
    )(x2, pos_embed[:S])
    return out2.reshape(S, B, D)


# speedup vs baseline: 3.8083x; 3.8083x over previous
import jax
import jax.numpy as jnp
from jax.experimental import pallas as pl


_BLK = 1024


def _add_kernel(x_ref, p_ref, o_ref):
    o_ref[...] = x_ref[...] + p_ref[...][:, None, :]


def kernel(x, pos_embed):
    S, B, D = x.shape
    blk = _BLK if S % _BLK == 0 else S
    return pl.pallas_call(
        _add_kernel,
        grid=(S // blk,),
        in_specs=[
            pl.BlockSpec((blk, B, D), lambda i: (i, 0, 0)),
            pl.BlockSpec((blk, D), lambda i: (i, 0)),
        ],
        out_specs=pl.BlockSpec((blk, B, D), lambda i: (i, 0, 0)),
        out_shape=jax.ShapeDtypeStruct((S, B, D), x.dtype),
    )(x, pos_embed[:S])
